# baseline (speedup 1.0000x reference)
import functools

import jax
import jax.numpy as jnp
from jax import lax
from jax.experimental import pallas as pl
from jax.experimental.pallas import tpu as pltpu
from jax.experimental.pallas import tpu_sc as plsc

_EMBED = 64
_B = 128


@functools.cache
def _make_lookup(seq: int, nb: int):
  info = plsc.get_sparse_core_info()
  nw = info.num_cores * info.num_subcores
  n_items = seq * nb
  items_per_w = n_items // nw
  assert n_items % nw == 0 and items_per_w % 2 == 0

  mesh = plsc.VectorSubcoreMesh(core_axis_name="c", subcore_axis_name="s")

  @functools.partial(
      pl.kernel,
      out_type=jax.ShapeDtypeStruct((seq, 8, nb, 8, _B), jnp.float32),
      mesh=mesh,
      scratch_types=[
          pltpu.VMEM((_B,), jnp.int32),
          pltpu.VMEM((_B,), jnp.int32),
          pltpu.VMEM((_B, _EMBED), jnp.float32),
          pltpu.VMEM((_B, _EMBED), jnp.float32),
          pltpu.VMEM((8, 8, _B), jnp.float32),
          pltpu.VMEM((8, 8, _B), jnp.float32),
          pltpu.SemaphoreType.DMA,
          pltpu.SemaphoreType.DMA,
          pltpu.SemaphoreType.DMA,
          pltpu.SemaphoreType.DMA,
          pltpu.SemaphoreType.DMA,
          pltpu.SemaphoreType.DMA,
      ],
      compiler_params=pltpu.CompilerParams(
          use_tc_tiling_on_sc=False, needs_layout_passes=False),
  )
  def lookup(idx_hbm, table_hbm, out_hbm,
             idx0, idx1, rows0, rows1, t0, t1,
             si0, si1, sg0, sg1, so0, so1):
    wid = lax.axis_index("s") * info.num_cores + lax.axis_index("c")
    base = wid * items_per_w

    iotas = [lax.iota(jnp.int32, 16) + 16 * j for j in range(8)]

    def item_sb(t):
      return t // nb, lax.rem(t, nb)

    def load_idx(t, idx_v, sem):
      s, b = item_sb(t)
      pltpu.async_copy(idx_hbm.at[s, b], idx_v, sem)

    def wait_idx(idx_v, sem):
      pltpu.make_async_copy(idx_hbm.at[0, 0], idx_v, sem).wait()

    def fire_gather(idx_v, rows_v, sem):
      pltpu.async_copy(table_hbm.at[idx_v], rows_v, sem)

    def wait_gather(rows_v, sem):
      pltpu.make_async_copy(table_hbm.at[pl.ds(0, _B)], rows_v, sem).wait()

    def transpose(rows_v, t_v):
      def body(e, carry):
        col = jnp.full((16,), e, jnp.int32)
        eh = e // 8
        el = lax.rem(e, 8)
        for j in range(8):
          v = plsc.load_gather(rows_v, [iotas[j], col])
          t_v[eh, el, pl.ds(16 * j, 16)] = v
        return carry
      lax.fori_loop(0, _EMBED, body, 0)

    def fire_out(t, t_v, sem):
      s, b = item_sb(t)
      for eh in range(8):
        pltpu.async_copy(t_v.at[eh], out_hbm.at[s, eh, b], sem)

    def wait_out(t_v, sem):
      pltpu.make_async_copy(t_v, out_hbm.at[0, :, 0], sem).wait()

    def load_idx_sync(t, idx_v):
      s, b = item_sb(t)
      pltpu.sync_copy(idx_hbm.at[s, b], idx_v)

    pltpu.sync_copy(idx_hbm.at[base // nb, lax.rem(base, nb)], idx0)
    fire_gather(idx0, rows0, sg0)
    pltpu.sync_copy(idx_hbm.at[(base + 1) // nb, lax.rem(base + 1, nb)], idx1)
    fire_gather(idx1, rows1, sg1)

    def slot(t, idx_v, rows_v, t_v, si, sg, so, fire_next, first):
      wait_gather(rows_v, sg)
      if not first:
        wait_out(t_v, so)
      transpose(rows_v, t_v)
      fire_out(t, t_v, so)
      if fire_next:
        load_idx_sync(t + 2, idx_v)
        fire_gather(idx_v, rows_v, sg)

    slot(base, idx0, rows0, t0, si0, sg0, so0, True, True)
    slot(base + 1, idx1, rows1, t1, si1, sg1, so1, True, True)

    def body2(g, carry):
      t = base + 2 * g
      slot(t, idx0, rows0, t0, si0, sg0, so0, True, False)
      slot(t + 1, idx1, rows1, t1, si1, sg1, so1, True, False)
      return carry

    lax.fori_loop(1, items_per_w // 2 - 1, body2, 0)

    t = base + items_per_w - 2
    slot(t, idx0, rows0, t0, si0, sg0, so0, False, False)
    slot(t + 1, idx1, rows1, t1, si1, sg1, so1, False, False)
    wait_out(t0, so0)
    wait_out(t1, so1)

  return lookup


def kernel(x, table):
  b, seq = x.shape
  nb = b // _B
  idx = x.astype(jnp.int32).T.reshape(seq, nb, _B)
  out5 = _make_lookup(seq, nb)(idx, table)
  return out5.transpose((2, 4, 0, 1, 3)).reshape(b, seq, _EMBED)


# speedup vs baseline: 1.4995x; 1.4995x over previous
import functools

import jax
import jax.numpy as jnp
from jax import lax
from jax.experimental import pallas as pl
from jax.experimental.pallas import tpu as pltpu
from jax.experimental.pallas import tpu_sc as plsc

_EMBED = 64
_B = 128


@functools.cache
def _make_lookup(seq: int, nb: int):
  info = plsc.get_sparse_core_info()
  nw = info.num_cores * info.num_subcores
  n_items = seq * nb
  items_per_w = n_items // nw
  assert n_items % nw == 0 and items_per_w % 2 == 0

  mesh = plsc.VectorSubcoreMesh(core_axis_name="c", subcore_axis_name="s")

  @functools.partial(
      pl.kernel,
      out_type=jax.ShapeDtypeStruct((seq, 8, nb, 8, _B), jnp.float32),
      mesh=mesh,
      scratch_types=[
          pltpu.VMEM((_B,), jnp.int32),
          pltpu.VMEM((_B,), jnp.int32),
          pltpu.VMEM((_B, _EMBED), jnp.float32),
          pltpu.VMEM((_B, _EMBED), jnp.float32),
          pltpu.VMEM((8, 8, _B), jnp.float32),
          pltpu.VMEM((8, 8, _B), jnp.float32),
          pltpu.SemaphoreType.DMA,
          pltpu.SemaphoreType.DMA,
          pltpu.SemaphoreType.DMA,
          pltpu.SemaphoreType.DMA,
          pltpu.SemaphoreType.DMA,
          pltpu.SemaphoreType.DMA,
      ],
      compiler_params=pltpu.CompilerParams(
          use_tc_tiling_on_sc=False, needs_layout_passes=False),
  )
  def lookup(idx_hbm, table_hbm, out_hbm,
             idx0, idx1, rows0, rows1, t0, t1,
             si0, si1, sg0, sg1, so0, so1):
    wid = lax.axis_index("s") * info.num_cores + lax.axis_index("c")
    base = wid * items_per_w

    iotas = [lax.iota(jnp.int32, 16) + 16 * j for j in range(8)]

    def item_sb(t):
      return t // nb, lax.rem(t, nb)

    def load_idx(t, idx_v, sem):
      s, b = item_sb(t)
      pltpu.async_copy(idx_hbm.at[s, b], idx_v, sem)

    def wait_idx(idx_v, sem):
      pltpu.make_async_copy(idx_hbm.at[0, 0], idx_v, sem).wait()

    def fire_gather(idx_v, rows_v, sem):
      pltpu.async_copy(table_hbm.at[idx_v], rows_v, sem)

    def wait_gather(rows_v, sem):
      pltpu.make_async_copy(table_hbm.at[pl.ds(0, _B)], rows_v, sem).wait()

    def transpose(rows_v, t_v):
      @plsc.parallel_loop(0, _EMBED, step=1, unroll=8)
      def _body(e):
        col = jnp.full((16,), e, jnp.int32)
        eh = e // 8
        el = lax.rem(e, 8)
        for j in range(8):
          v = plsc.load_gather(rows_v, [iotas[j], col])
          t_v[eh, el, pl.ds(16 * j, 16)] = v

    def fire_out(t, t_v, sem):
      s, b = item_sb(t)
      for eh in range(8):
        pltpu.async_copy(t_v.at[eh], out_hbm.at[s, eh, b], sem)

    def wait_out(t_v, sem):
      pltpu.make_async_copy(t_v, out_hbm.at[0, :, 0], sem).wait()

    def load_idx_sync(t, idx_v):
      s, b = item_sb(t)
      pltpu.sync_copy(idx_hbm.at[s, b], idx_v)

    pltpu.sync_copy(idx_hbm.at[base // nb, lax.rem(base, nb)], idx0)
    fire_gather(idx0, rows0, sg0)
    pltpu.sync_copy(idx_hbm.at[(base + 1) // nb, lax.rem(base + 1, nb)], idx1)
    fire_gather(idx1, rows1, sg1)

    def slot(t, idx_v, rows_v, t_v, si, sg, so, fire_next, first):
      wait_gather(rows_v, sg)
      if fire_next:
        load_idx(t + 2, idx_v, si)
      if not first:
        wait_out(t_v, so)
      transpose(rows_v, t_v)
      fire_out(t, t_v, so)
      if fire_next:
        wait_idx(idx_v, si)
        fire_gather(idx_v, rows_v, sg)

    slot(base, idx0, rows0, t0, si0, sg0, so0, True, True)
    slot(base + 1, idx1, rows1, t1, si1, sg1, so1, True, True)

    def body2(g, carry):
      t = base + 2 * g
      slot(t, idx0, rows0, t0, si0, sg0, so0, True, False)
      slot(t + 1, idx1, rows1, t1, si1, sg1, so1, True, False)
      return carry

    lax.fori_loop(1, items_per_w // 2 - 1, body2, 0)

    t = base + items_per_w - 2
    slot(t, idx0, rows0, t0, si0, sg0, so0, False, False)
    slot(t + 1, idx1, rows1, t1, si1, sg1, so1, False, False)
    wait_out(t0, so0)
    wait_out(t1, so1)

  return lookup


def kernel(x, table):
  b, seq = x.shape
  nb = b // _B
  idx = x.astype(jnp.int32).T.reshape(seq, nb, _B)
  out5 = _make_lookup(seq, nb)(idx, table)
  return out5.transpose((2, 4, 0, 1, 3)).reshape(b, seq, _EMBED)
